# baseline (speedup 1.0000x reference)
import functools
import jax
import jax.numpy as jnp
from jax import lax
from jax.experimental import pallas as pl
from jax.experimental.pallas import tpu as pltpu
from jax.experimental.pallas import tpu_sc as plsc

T = 2048
H = 768
E = 64
K = 8
G = 8
TG = 4
I = 256
SI = 256
RSF = 2.5
BM = 128
NBLK = (T * K + E * BM) // BM
NEG = -1e30
BIGF = 1e9


def _sigmoid(x):
    return 1.0 / (1.0 + jnp.exp(-x))


def _gate_kernel(x_ref, wg_ref, bias_ref, comb_ref, d_ref, w_ref, cnt_ref):
    x = x_ref[...]
    logits = lax.dot_general(x, wg_ref[...], (((1,), (1,)), ((), ())),
                             preferred_element_type=jnp.float32)
    scores = _sigmoid(logits)
    sr = scores + bias_ref[...]

    iota64 = lax.broadcasted_iota(jnp.int32, (T, E), 1).astype(jnp.float32)
    iota8 = iota64[:, :G]

    gs_cols = []
    for g in range(G):
        sub = sr[:, g * (E // G):(g + 1) * (E // G)]
        subi = iota64[:, :E // G]
        m1 = jnp.max(sub, axis=1, keepdims=True)
        a1 = jnp.min(jnp.where(sub == m1, subi, BIGF), axis=1, keepdims=True)
        sub2 = jnp.where(subi == a1, NEG, sub)
        m2 = jnp.max(sub2, axis=1, keepdims=True)
        gs_cols.append(m1 + m2)
    gs = jnp.concatenate(gs_cols, axis=1)

    gmask = jnp.zeros((T, G), jnp.float32)
    work = gs
    for _ in range(TG):
        m = jnp.max(work, axis=1, keepdims=True)
        a = jnp.min(jnp.where(work == m, iota8, BIGF), axis=1, keepdims=True)
        pick = (iota8 == a).astype(jnp.float32)
        gmask = gmask + pick
        work = jnp.where(pick > 0, NEG, work)

    score_mask = jnp.concatenate(
        [jnp.broadcast_to(gmask[:, g:g + 1], (T, E // G)) for g in range(G)],
        axis=1)
    masked = jnp.where(score_mask > 0, sr, NEG)

    picks = []
    work2 = masked
    sel = jnp.zeros((T, E), jnp.float32)
    for _ in range(K):
        m = jnp.max(work2, axis=1, keepdims=True)
        a = jnp.min(jnp.where(work2 == m, iota64, BIGF), axis=1, keepdims=True)
        pick = (iota64 == a).astype(jnp.float32)
        picks.append(pick)
        sel = sel + pick
        work2 = jnp.where(pick > 0, NEG, work2)

    ssum = jnp.sum(sel * scores, axis=1, keepdims=True)
    scale = RSF / (ssum + 1e-20)
    comb_ref[...] = sel * scores * scale

    CT = 256
    tri = (lax.broadcasted_iota(jnp.int32, (CT, CT), 0) >=
           lax.broadcasted_iota(jnp.int32, (CT, CT), 1)).astype(jnp.float32)
    off = jnp.zeros((1, E), jnp.float32)
    pos_chunks = []
    for c in range(T // CT):
        seg = sel[c * CT:(c + 1) * CT, :]
        cs = lax.dot_general(tri, seg, (((1,), (0,)), ((), ())),
                             preferred_element_type=jnp.float32) + off
        pos_chunks.append(cs)
        off = cs[CT - 1:CT, :]
    posincl = jnp.concatenate(pos_chunks, axis=0)
    counts = off
    pc = jnp.floor((counts + (BM - 1)) * (1.0 / BM)) * BM
    iu_r = lax.broadcasted_iota(jnp.int32, (E, E), 0)
    iu_c = lax.broadcasted_iota(jnp.int32, (E, E), 1)
    su = (iu_r < iu_c).astype(jnp.float32)
    pexcl = lax.dot_general(pc, su, (((1,), (0,)), ((), ())),
                            preferred_element_type=jnp.float32)
    dmat = pexcl + posincl - 1.0

    w_cols = []
    d_cols = []
    for k in range(K):
        pick = picks[k]
        w_cols.append(jnp.sum(pick * scores, axis=1, keepdims=True))
        d_cols.append(jnp.sum(pick * dmat, axis=1, keepdims=True))
    w_ref[...] = jnp.concatenate(w_cols, axis=1) * scale
    d_ref[...] = jnp.concatenate(d_cols, axis=1).astype(jnp.int32)
    cnt_ref[...] = counts


def _gate(x, Wg, expert_bias):
    return pl.pallas_call(
        _gate_kernel,
        out_shape=(
            jax.ShapeDtypeStruct((T, E), jnp.float32),
            jax.ShapeDtypeStruct((T, K), jnp.int32),
            jax.ShapeDtypeStruct((T, K), jnp.float32),
            jax.ShapeDtypeStruct((1, E), jnp.float32),
        ),
    )(x, Wg, expert_bias.reshape(1, E))


def _dense_kernel(x_ref, comb_ref, w1_ref, w3_ref, w2_ref,
                  sw1_ref, sw3_ref, sw2_ref, out_ref, acc_ref):
    e = pl.program_id(0)
    x = x_ref[...]

    @pl.when(e == 0)
    def _():
        acc_ref[...] = jnp.zeros_like(acc_ref)

    g = lax.dot_general(x, w1_ref[0], (((1,), (1,)), ((), ())),
                        preferred_element_type=jnp.float32)
    u = lax.dot_general(x, w3_ref[0], (((1,), (1,)), ((), ())),
                        preferred_element_type=jnp.float32)
    h = g * _sigmoid(g) * u
    o = lax.dot_general(h, w2_ref[0], (((1,), (1,)), ((), ())),
                        preferred_element_type=jnp.float32)
    colid = lax.broadcasted_iota(jnp.int32, (T, E), 1)
    ce = jnp.sum(jnp.where(colid == e, comb_ref[...], 0.0),
                 axis=1, keepdims=True)
    acc_ref[...] += ce * o

    @pl.when(e == E - 1)
    def _():
        sg = lax.dot_general(x, sw1_ref[...], (((1,), (1,)), ((), ())),
                             preferred_element_type=jnp.float32)
        su_ = lax.dot_general(x, sw3_ref[...], (((1,), (1,)), ((), ())),
                              preferred_element_type=jnp.float32)
        sh = sg * _sigmoid(sg) * su_
        so = lax.dot_general(sh, sw2_ref[...], (((1,), (1,)), ((), ())),
                             preferred_element_type=jnp.float32)
        out_ref[...] = acc_ref[...] + so


NC = 2
NS = 16
NW = NC * NS
PAIRS = T * K
PPW = PAIRS // NW
CH = 64
NCH = PPW // CH
CHC = 32
NCHC = PPW // CHC
S = NBLK * BM
HV = H // 16


def _worker_id():
    return lax.axis_index("s") * NC + lax.axis_index("c")


def _dispatch_body(x_hbm, d_hbm, xs_hbm, tok_v, dv_v, buf, sem_g, sem_s):
    base0 = _worker_id() * PPW
    lanes = lax.iota(jnp.int32, 16)

    def chunk(c, carry):
        base = pl.multiple_of(base0 + c * CH, CH)
        for j in range(CH // 16):
            tok_v[pl.ds(j * 16, 16)] = lax.shift_right_logical(
                base + j * 16 + lanes, 3)
        pltpu.sync_copy(d_hbm.at[pl.ds(base, CH)], dv_v)
        pltpu.async_copy(x_hbm.at[tok_v], buf, sem_g).wait()
        pltpu.async_copy(buf, xs_hbm.at[dv_v], sem_s).wait()
        return carry

    lax.fori_loop(0, NCH, chunk, 0)


_dispatch = pl.kernel(
    _dispatch_body,
    out_type=jax.ShapeDtypeStruct((S, H), jnp.float32),
    mesh=plsc.VectorSubcoreMesh(core_axis_name="c", subcore_axis_name="s"),
    compiler_params=pltpu.CompilerParams(needs_layout_passes=False),
    scratch_types=[
        pltpu.VMEM((CH,), jnp.int32),
        pltpu.VMEM((CH,), jnp.int32),
        pltpu.VMEM((CH, H), jnp.float32),
        pltpu.SemaphoreType.DMA,
        pltpu.SemaphoreType.DMA,
    ],
)


def _combine_body(ys_hbm, d_hbm, w_hbm, out_hbm, dv_v, wv, buf, acc, sem_g):
    wid = _worker_id()
    base0 = wid * PPW
    lanes = lax.iota(jnp.int32, 16)
    zeros16 = jnp.zeros((16,), jnp.float32)
    TPW = PPW // K

    def zrow(r, c):
        for j in range(HV):
            acc[r, pl.ds(j * 16, 16)] = zeros16
        return c

    lax.fori_loop(0, TPW, zrow, 0)

    def chunk(c, carry):
        base = pl.multiple_of(base0 + c * CH, CH)
        pltpu.sync_copy(d_hbm.at[pl.ds(base, CH)], dv_v)
        pltpu.sync_copy(w_hbm.at[pl.ds(base, CH)], wv)
        pltpu.async_copy(ys_hbm.at[dv_v], buf, sem_g).wait()

        def grp(rg, cc):
            w16 = wv[pl.ds(rg * 16, 16)]
            for i in range(16):
                wi = jnp.sum(jnp.where(lanes == i, w16, 0.0))
                r = rg * 16 + i
                tloc = lax.shift_right_logical(c * CH + rg * 16 + i, 3)
                for j in range(HV):
                    plsc.addupdate(acc.at[tloc, pl.ds(j * 16, 16)],
                                   buf[r, pl.ds(j * 16, 16)] * wi)
            return cc

        lax.fori_loop(0, CH // 16, grp, 0)
        return carry

    lax.fori_loop(0, NCH, chunk, 0)
    pltpu.sync_copy(acc, out_hbm.at[pl.ds(wid * TPW, TPW)])


_combine = pl.kernel(
    _combine_body,
    out_type=jax.ShapeDtypeStruct((T, H), jnp.float32),
    mesh=plsc.VectorSubcoreMesh(core_axis_name="c", subcore_axis_name="s"),
    compiler_params=pltpu.CompilerParams(needs_layout_passes=False),
    scratch_types=[
        pltpu.VMEM((CH,), jnp.int32),
        pltpu.VMEM((CH,), jnp.float32),
        pltpu.VMEM((CH, H), jnp.float32),
        pltpu.VMEM((PPW // K, H), jnp.float32),
        pltpu.SemaphoreType.DMA,
    ],
)


def _mlp_kernel(be_ref, rf_ref, xs_ref, w1_ref, w3_ref, w2_ref, ys_ref):
    b = pl.program_id(0)

    @pl.when(rf_ref[b] == 1)
    def _():
        xb = xs_ref[...]
        g = lax.dot_general(xb, w1_ref[0], (((1,), (1,)), ((), ())),
                            preferred_element_type=jnp.float32)
        u = lax.dot_general(xb, w3_ref[0], (((1,), (1,)), ((), ())),
                            preferred_element_type=jnp.float32)
        h = g * _sigmoid(g) * u
        ys_ref[...] = lax.dot_general(h, w2_ref[0], (((1,), (1,)), ((), ())),
                                      preferred_element_type=jnp.float32)


def _mlp(xs, w1, w3, w2, be, rf):
    return pl.pallas_call(
        _mlp_kernel,
        grid_spec=pltpu.PrefetchScalarGridSpec(
            num_scalar_prefetch=2,
            grid=(NBLK,),
            in_specs=[
                pl.BlockSpec((BM, H), lambda b, be, rf: (b, 0)),
                pl.BlockSpec((1, I, H), lambda b, be, rf: (be[b], 0, 0)),
                pl.BlockSpec((1, I, H), lambda b, be, rf: (be[b], 0, 0)),
                pl.BlockSpec((1, H, I), lambda b, be, rf: (be[b], 0, 0)),
            ],
            out_specs=pl.BlockSpec((BM, H), lambda b, be, rf: (b, 0)),
        ),
        out_shape=jax.ShapeDtypeStruct((S, H), jnp.float32),
        compiler_params=pltpu.CompilerParams(
            dimension_semantics=("arbitrary",)),
    )(be, rf, xs, w1, w3, w2)


def _final_kernel(x_ref, sw1_ref, sw3_ref, sw2_ref, p0_ref, out_ref):
    xb = x_ref[...]
    sg = lax.dot_general(xb, sw1_ref[...], (((1,), (1,)), ((), ())),
                         preferred_element_type=jnp.float32)
    su_ = lax.dot_general(xb, sw3_ref[...], (((1,), (1,)), ((), ())),
                          preferred_element_type=jnp.float32)
    sh = sg * _sigmoid(sg) * su_
    so = lax.dot_general(sh, sw2_ref[...], (((1,), (1,)), ((), ())),
                         preferred_element_type=jnp.float32)
    out_ref[...] = so + p0_ref[...]


def _final(x, sw1, sw3, sw2, part):
    BT = 256
    return pl.pallas_call(
        _final_kernel,
        grid=(T // BT,),
        in_specs=[
            pl.BlockSpec((BT, H), lambda t: (t, 0)),
            pl.BlockSpec((SI, H), lambda t: (0, 0)),
            pl.BlockSpec((SI, H), lambda t: (0, 0)),
            pl.BlockSpec((H, SI), lambda t: (0, 0)),
            pl.BlockSpec((BT, H), lambda t: (t, 0)),
        ],
        out_specs=pl.BlockSpec((BT, H), lambda t: (t, 0)),
        out_shape=jax.ShapeDtypeStruct((T, H), jnp.float32),
        compiler_params=pltpu.CompilerParams(
            dimension_semantics=("arbitrary",)),
    )(x, sw1, sw3, sw2, part)


def kernel(hidden_states, image_mask, audio_mask, Wg, expert_bias,
           w1, w3, w2, sw1, sw3, sw2):
    x = hidden_states.reshape(-1, H)
    combine, d_tk, w_tk, counts = _gate(x, Wg, expert_bias)

    cnt = counts[0].astype(jnp.int32)
    nblk_e = (cnt + BM - 1) // BM
    ends = jnp.cumsum(nblk_e)
    total_blocks = ends[E - 1]
    bids = jnp.arange(NBLK, dtype=jnp.int32)
    be = jnp.minimum(
        jnp.sum((bids[:, None] >= ends[None, :]).astype(jnp.int32), axis=1),
        E - 1).astype(jnp.int32)
    rf = (bids < total_blocks).astype(jnp.int32)

    dflat = d_tk.reshape(PAIRS)
    wflat = w_tk.reshape(PAIRS)

    xs = _dispatch(x, dflat)
    ys = _mlp(xs, w1, w3, w2, be, rf)
    routed = _combine(ys, dflat, wflat)
    return _final(x, sw1, sw3, sw2, routed)


# speedup vs baseline: 1.2960x; 1.2960x over previous
import functools
import jax
import jax.numpy as jnp
from jax import lax
from jax.experimental import pallas as pl
from jax.experimental.pallas import tpu as pltpu
from jax.experimental.pallas import tpu_sc as plsc

T = 2048
H = 768
E = 64
K = 8
G = 8
TG = 4
I = 256
SI = 256
RSF = 2.5
BM = 128
NBLK = (T * K + E * BM) // BM
NEG = -1e30
BIGF = 1e9


def _sigmoid(x):
    return 1.0 / (1.0 + jnp.exp(-x))


def _gate_kernel(x_ref, wg_ref, bias_ref, comb_ref, d_ref, w_ref, cnt_ref):
    x = x_ref[...]
    logits = lax.dot_general(x, wg_ref[...], (((1,), (1,)), ((), ())),
                             preferred_element_type=jnp.float32)
    scores = _sigmoid(logits)
    sr = scores + bias_ref[...]

    iota64 = lax.broadcasted_iota(jnp.int32, (T, E), 1).astype(jnp.float32)
    iota8 = iota64[:, :G]

    gs_cols = []
    for g in range(G):
        sub = sr[:, g * (E // G):(g + 1) * (E // G)]
        subi = iota64[:, :E // G]
        m1 = jnp.max(sub, axis=1, keepdims=True)
        a1 = jnp.min(jnp.where(sub == m1, subi, BIGF), axis=1, keepdims=True)
        sub2 = jnp.where(subi == a1, NEG, sub)
        m2 = jnp.max(sub2, axis=1, keepdims=True)
        gs_cols.append(m1 + m2)
    gs = jnp.concatenate(gs_cols, axis=1)

    gmask = jnp.zeros((T, G), jnp.float32)
    work = gs
    for _ in range(TG):
        m = jnp.max(work, axis=1, keepdims=True)
        a = jnp.min(jnp.where(work == m, iota8, BIGF), axis=1, keepdims=True)
        pick = (iota8 == a).astype(jnp.float32)
        gmask = gmask + pick
        work = jnp.where(pick > 0, NEG, work)

    score_mask = jnp.concatenate(
        [jnp.broadcast_to(gmask[:, g:g + 1], (T, E // G)) for g in range(G)],
        axis=1)
    masked = jnp.where(score_mask > 0, sr, NEG)

    picks = []
    work2 = masked
    sel = jnp.zeros((T, E), jnp.float32)
    for _ in range(K):
        m = jnp.max(work2, axis=1, keepdims=True)
        a = jnp.min(jnp.where(work2 == m, iota64, BIGF), axis=1, keepdims=True)
        pick = (iota64 == a).astype(jnp.float32)
        picks.append(pick)
        sel = sel + pick
        work2 = jnp.where(pick > 0, NEG, work2)

    ssum = jnp.sum(sel * scores, axis=1, keepdims=True)
    scale = RSF / (ssum + 1e-20)
    comb_ref[...] = sel * scores * scale

    CT = 256
    tri = (lax.broadcasted_iota(jnp.int32, (CT, CT), 0) >=
           lax.broadcasted_iota(jnp.int32, (CT, CT), 1)).astype(jnp.float32)
    off = jnp.zeros((1, E), jnp.float32)
    pos_chunks = []
    for c in range(T // CT):
        seg = sel[c * CT:(c + 1) * CT, :]
        cs = lax.dot_general(tri, seg, (((1,), (0,)), ((), ())),
                             preferred_element_type=jnp.float32) + off
        pos_chunks.append(cs)
        off = cs[CT - 1:CT, :]
    posincl = jnp.concatenate(pos_chunks, axis=0)
    counts = off
    pc = jnp.floor((counts + (BM - 1)) * (1.0 / BM)) * BM
    iu_r = lax.broadcasted_iota(jnp.int32, (E, E), 0)
    iu_c = lax.broadcasted_iota(jnp.int32, (E, E), 1)
    su = (iu_r < iu_c).astype(jnp.float32)
    pexcl = lax.dot_general(pc, su, (((1,), (0,)), ((), ())),
                            preferred_element_type=jnp.float32)
    dmat = pexcl + posincl - 1.0

    w_cols = []
    d_cols = []
    for k in range(K):
        pick = picks[k]
        w_cols.append(jnp.sum(pick * scores, axis=1, keepdims=True))
        d_cols.append(jnp.sum(pick * dmat, axis=1, keepdims=True))
    w_ref[...] = jnp.concatenate(w_cols, axis=1) * scale
    d_ref[...] = jnp.concatenate(d_cols, axis=1).astype(jnp.int32)
    cnt_ref[...] = counts


def _gate(x, Wg, expert_bias):
    return pl.pallas_call(
        _gate_kernel,
        out_shape=(
            jax.ShapeDtypeStruct((T, E), jnp.float32),
            jax.ShapeDtypeStruct((T, K), jnp.int32),
            jax.ShapeDtypeStruct((T, K), jnp.float32),
            jax.ShapeDtypeStruct((1, E), jnp.float32),
        ),
    )(x, Wg, expert_bias.reshape(1, E))


def _dense_kernel(x_ref, comb_ref, w1_ref, w3_ref, w2_ref,
                  sw1_ref, sw3_ref, sw2_ref, out_ref, acc_ref):
    e = pl.program_id(0)
    x = x_ref[...]

    @pl.when(e == 0)
    def _():
        acc_ref[...] = jnp.zeros_like(acc_ref)

    g = lax.dot_general(x, w1_ref[0], (((1,), (1,)), ((), ())),
                        preferred_element_type=jnp.float32)
    u = lax.dot_general(x, w3_ref[0], (((1,), (1,)), ((), ())),
                        preferred_element_type=jnp.float32)
    h = g * _sigmoid(g) * u
    o = lax.dot_general(h, w2_ref[0], (((1,), (1,)), ((), ())),
                        preferred_element_type=jnp.float32)
    colid = lax.broadcasted_iota(jnp.int32, (T, E), 1)
    ce = jnp.sum(jnp.where(colid == e, comb_ref[...], 0.0),
                 axis=1, keepdims=True)
    acc_ref[...] += ce * o

    @pl.when(e == E - 1)
    def _():
        sg = lax.dot_general(x, sw1_ref[...], (((1,), (1,)), ((), ())),
                             preferred_element_type=jnp.float32)
        su_ = lax.dot_general(x, sw3_ref[...], (((1,), (1,)), ((), ())),
                              preferred_element_type=jnp.float32)
        sh = sg * _sigmoid(sg) * su_
        so = lax.dot_general(sh, sw2_ref[...], (((1,), (1,)), ((), ())),
                             preferred_element_type=jnp.float32)
        out_ref[...] = acc_ref[...] + so


NC = 2
NS = 16
NW = NC * NS
PAIRS = T * K
PPW = PAIRS // NW
CH = 64
NCH = PPW // CH
CHC = 32
NCHC = PPW // CHC
S = NBLK * BM
HV = H // 16


def _worker_id():
    return lax.axis_index("s") * NC + lax.axis_index("c")


def _dispatch_body(x_hbm, d_hbm, xs_hbm, tok_v, dv_v, buf, sem_g, sem_s):
    base0 = _worker_id() * PPW
    lanes = lax.iota(jnp.int32, 16)

    def chunk(c, carry):
        base = pl.multiple_of(base0 + c * CH, CH)
        for j in range(CH // 16):
            tok_v[pl.ds(j * 16, 16)] = lax.shift_right_logical(
                base + j * 16 + lanes, 3)
        pltpu.sync_copy(d_hbm.at[pl.ds(base, CH)], dv_v)
        pltpu.async_copy(x_hbm.at[tok_v], buf, sem_g).wait()
        pltpu.async_copy(buf, xs_hbm.at[dv_v], sem_s).wait()
        return carry

    lax.fori_loop(0, NCH, chunk, 0)


_dispatch = pl.kernel(
    _dispatch_body,
    out_type=jax.ShapeDtypeStruct((S, H), jnp.float32),
    mesh=plsc.VectorSubcoreMesh(core_axis_name="c", subcore_axis_name="s"),
    compiler_params=pltpu.CompilerParams(needs_layout_passes=False),
    scratch_types=[
        pltpu.VMEM((CH,), jnp.int32),
        pltpu.VMEM((CH,), jnp.int32),
        pltpu.VMEM((CH, H), jnp.float32),
        pltpu.SemaphoreType.DMA,
        pltpu.SemaphoreType.DMA,
    ],
)


def _combine_body(ys_hbm, d_hbm, pairs_hbm, dv_v, buf, sem_g, sem_s):
    base0 = _worker_id() * PPW

    def chunk(c, carry):
        base = pl.multiple_of(base0 + c * CH, CH)
        pltpu.sync_copy(d_hbm.at[pl.ds(base, CH)], dv_v)
        pltpu.async_copy(ys_hbm.at[dv_v], buf, sem_g).wait()
        pltpu.async_copy(buf, pairs_hbm.at[pl.ds(base, CH)], sem_s).wait()
        return carry

    lax.fori_loop(0, NCH, chunk, 0)


_combine = pl.kernel(
    _combine_body,
    out_type=jax.ShapeDtypeStruct((PAIRS, H), jnp.float32),
    mesh=plsc.VectorSubcoreMesh(core_axis_name="c", subcore_axis_name="s"),
    compiler_params=pltpu.CompilerParams(needs_layout_passes=False),
    scratch_types=[
        pltpu.VMEM((CH,), jnp.int32),
        pltpu.VMEM((CH, H), jnp.float32),
        pltpu.SemaphoreType.DMA,
        pltpu.SemaphoreType.DMA,
    ],
)


def _mlp_kernel(be_ref, rf_ref, xs_ref, w1_ref, w3_ref, w2_ref, ys_ref):
    b = pl.program_id(0)

    @pl.when(rf_ref[b] == 1)
    def _():
        xb = xs_ref[...]
        g = lax.dot_general(xb, w1_ref[0], (((1,), (1,)), ((), ())),
                            preferred_element_type=jnp.float32)
        u = lax.dot_general(xb, w3_ref[0], (((1,), (1,)), ((), ())),
                            preferred_element_type=jnp.float32)
        h = g * _sigmoid(g) * u
        ys_ref[...] = lax.dot_general(h, w2_ref[0], (((1,), (1,)), ((), ())),
                                      preferred_element_type=jnp.float32)


def _mlp(xs, w1, w3, w2, be, rf):
    return pl.pallas_call(
        _mlp_kernel,
        grid_spec=pltpu.PrefetchScalarGridSpec(
            num_scalar_prefetch=2,
            grid=(NBLK,),
            in_specs=[
                pl.BlockSpec((BM, H), lambda b, be, rf: (b, 0)),
                pl.BlockSpec((1, I, H), lambda b, be, rf: (be[b], 0, 0)),
                pl.BlockSpec((1, I, H), lambda b, be, rf: (be[b], 0, 0)),
                pl.BlockSpec((1, H, I), lambda b, be, rf: (be[b], 0, 0)),
            ],
            out_specs=pl.BlockSpec((BM, H), lambda b, be, rf: (b, 0)),
        ),
        out_shape=jax.ShapeDtypeStruct((S, H), jnp.float32),
        compiler_params=pltpu.CompilerParams(
            dimension_semantics=("arbitrary",)),
    )(be, rf, xs, w1, w3, w2)


FBT = 128


def _final_kernel(x_ref, w_ref, pairs_ref, sw1_ref, sw3_ref, sw2_ref,
                  out_ref):
    w = w_ref[...]
    r = lax.broadcasted_iota(jnp.int32, (FBT, FBT * K), 0)
    c = lax.broadcasted_iota(jnp.int32, (FBT, FBT * K), 1)
    wsel = jnp.zeros((FBT, FBT * K), jnp.float32)
    for k in range(K):
        wsel = wsel + jnp.where(c == r * K + k, w[:, k:k + 1], 0.0)
    routed = lax.dot_general(wsel, pairs_ref[...], (((1,), (0,)), ((), ())),
                             preferred_element_type=jnp.float32)

    xb = x_ref[...]
    sg = lax.dot_general(xb, sw1_ref[...], (((1,), (1,)), ((), ())),
                         preferred_element_type=jnp.float32)
    su_ = lax.dot_general(xb, sw3_ref[...], (((1,), (1,)), ((), ())),
                          preferred_element_type=jnp.float32)
    sh = sg * _sigmoid(sg) * su_
    so = lax.dot_general(sh, sw2_ref[...], (((1,), (1,)), ((), ())),
                         preferred_element_type=jnp.float32)
    out_ref[...] = so + routed


def _final(x, w_tk, pairs, sw1, sw3, sw2):
    return pl.pallas_call(
        _final_kernel,
        grid=(T // FBT,),
        in_specs=[
            pl.BlockSpec((FBT, H), lambda t: (t, 0)),
            pl.BlockSpec((FBT, K), lambda t: (t, 0)),
            pl.BlockSpec((FBT * K, H), lambda t: (t, 0)),
            pl.BlockSpec((SI, H), lambda t: (0, 0)),
            pl.BlockSpec((SI, H), lambda t: (0, 0)),
            pl.BlockSpec((H, SI), lambda t: (0, 0)),
        ],
        out_specs=pl.BlockSpec((FBT, H), lambda t: (t, 0)),
        out_shape=jax.ShapeDtypeStruct((T, H), jnp.float32),
        compiler_params=pltpu.CompilerParams(
            dimension_semantics=("arbitrary",)),
    )(x, w_tk, pairs, sw1, sw3, sw2)


def kernel(hidden_states, image_mask, audio_mask, Wg, expert_bias,
           w1, w3, w2, sw1, sw3, sw2):
    x = hidden_states.reshape(-1, H)
    combine, d_tk, w_tk, counts = _gate(x, Wg, expert_bias)

    cnt = counts[0].astype(jnp.int32)
    nblk_e = (cnt + BM - 1) // BM
    ends = jnp.cumsum(nblk_e)
    total_blocks = ends[E - 1]
    bids = jnp.arange(NBLK, dtype=jnp.int32)
    be = jnp.minimum(
        jnp.sum((bids[:, None] >= ends[None, :]).astype(jnp.int32), axis=1),
        E - 1).astype(jnp.int32)
    rf = (bids < total_blocks).astype(jnp.int32)

    dflat = d_tk.reshape(PAIRS)
    wflat = w_tk.reshape(PAIRS)

    xs = _dispatch(x, dflat)
    ys = _mlp(xs, w1, w3, w2, be, rf)
    pairs = _combine(ys, dflat)
    return _final(x, w_tk, pairs, sw1, sw3, sw2)
